# baseline (speedup 1.0000x reference)
import functools

import jax
import jax.numpy as jnp
from jax.experimental import pallas as pl
from jax.experimental.pallas import tpu as pltpu

_N = 50000
_E = 800000
_H = 160
_R = 400


def _leaky(x):
    return jnp.where(x >= 0, x, 0.01 * x)


def _full(shape):
    return pl.BlockSpec(shape, lambda i: tuple(0 for _ in shape))


def _dense_body(np_ref, nc_ref, des_ref, tw_ref, px_ref,
                Wnp, bnp, Wnc, bnc, Wdes, bdes, Wtx, btx, Wtw, btw,
                Win, bin_, h_ref):
    n = _leaky(np_ref[...] @ Wnp[...].T + bnp[...])
    c = _leaky(nc_ref[...] @ Wnc[...].T + bnc[...])
    d = _leaky(des_ref[...] @ Wdes[...].T + bdes[...])
    tw = _leaky(tw_ref[...] @ Wtx[...].T + btx[...])
    pt = _leaky(px_ref[...] @ Wtw[...].T + btw[...])
    h = jnp.concatenate([n, c, d, tw, pt], axis=1)
    h_ref[...] = _leaky(h @ Win[...].T + bin_[...])


def _dense_stage(num_prop, num_category, des_tensor, tweet_tensor, pre_x, p):
    grid = (_N // _R,)
    row = lambda shape: pl.BlockSpec((_R,) + shape[1:],
                                     lambda i: (i,) + tuple(0 for _ in shape[1:]))
    in_specs = [
        row((_N, 6)), row((_N, 11)), row((_N, 768)), row((_N, 768)), row((_N, 768)),
        _full((32, 6)), _full((1, 32)),
        _full((32, 11)), _full((1, 32)),
        _full((32, 768)), _full((1, 32)),
        _full((32, 768)), _full((1, 32)),
        _full((32, 768)), _full((1, 32)),
        _full((_H, _H)), _full((1, _H)),
    ]
    return pl.pallas_call(
        _dense_body,
        grid=grid,
        in_specs=in_specs,
        out_specs=pl.BlockSpec((_R, _H), lambda i: (i, 0)),
        out_shape=jax.ShapeDtypeStruct((_N, _H), jnp.float32),
    )(num_prop, num_category, des_tensor, tweet_tensor, pre_x,
      p['W_np'], p['b_np'].reshape(1, 32),
      p['W_nc'], p['b_nc'].reshape(1, 32),
      p['W_des'], p['b_des'].reshape(1, 32),
      p['W_text'], p['b_text'].reshape(1, 32),
      p['W_tweet'], p['b_tweet'].reshape(1, 32),
      p['W_in'], p['b_in'].reshape(1, _H))


def _combine_body(agg_ref, cnt_ref, h_ref, Wl, bl, Wr, out_ref):
    agg = agg_ref[...] / jnp.maximum(cnt_ref[...], 1.0)
    out_ref[...] = agg @ Wl[...].T + bl[...] + h_ref[...] @ Wr[...].T


def _combine_stage(agg, cnt, h, Wl, bl, Wr):
    grid = (_N // _R,)
    return pl.pallas_call(
        _combine_body,
        grid=grid,
        in_specs=[
            pl.BlockSpec((_R, _H), lambda i: (i, 0)),
            pl.BlockSpec((_R, 1), lambda i: (i, 0)),
            pl.BlockSpec((_R, _H), lambda i: (i, 0)),
            _full((_H, _H)), _full((1, _H)), _full((_H, _H)),
        ],
        out_specs=pl.BlockSpec((_R, _H), lambda i: (i, 0)),
        out_shape=jax.ShapeDtypeStruct((_N, _H), jnp.float32),
    )(agg, cnt, h, Wl, bl.reshape(1, _H), Wr)


def _out_body(h_ref, Wo1, bo1, Wo2, bo2, out_ref, em_ref):
    em = _leaky(h_ref[...] @ Wo1[...].T + bo1[...])
    em_ref[...] = em
    out_ref[...] = em @ Wo2[...].T + bo2[...]


def _out_stage(h, p):
    grid = (_N // _R,)
    return pl.pallas_call(
        _out_body,
        grid=grid,
        in_specs=[
            pl.BlockSpec((_R, _H), lambda i: (i, 0)),
            _full((80, _H)), _full((1, 80)),
            _full((2, 80)), _full((1, 2)),
        ],
        out_specs=[
            pl.BlockSpec((_R, 2), lambda i: (i, 0)),
            pl.BlockSpec((_R, 80), lambda i: (i, 0)),
        ],
        out_shape=[
            jax.ShapeDtypeStruct((_N, 2), jnp.float32),
            jax.ShapeDtypeStruct((_N, 80), jnp.float32),
        ],
    )(h, p['W_o1'], p['b_o1'].reshape(1, 80), p['W_o2'], p['b_o2'].reshape(1, 2))


def kernel(pre_x, x, num_prop, num_category, des_tensor, tweet_tensor,
           edge_index, edge_type, params):
    p = params
    src = edge_index[0]
    dst = edge_index[1]

    h = _dense_stage(num_prop, num_category, des_tensor, tweet_tensor, pre_x, p)

    cnt = jax.ops.segment_sum(jnp.ones((_E,), jnp.float32), dst,
                              num_segments=_N).reshape(_N, 1)

    msgs = jnp.take(h, src, axis=0)
    agg = jax.ops.segment_sum(msgs, dst, num_segments=_N)
    h1 = _combine_stage(agg, cnt, h, p['W1l'], p['b1l'], p['W1r'])

    msgs = jnp.take(h1, src, axis=0)
    agg = jax.ops.segment_sum(msgs, dst, num_segments=_N)
    h2 = _combine_stage(agg, cnt, h1, p['W2l'], p['b2l'], p['W2r'])

    out, em = _out_stage(h2, p)
    return (out, em)


# speedup vs baseline: 3.9924x; 3.9924x over previous
import functools

import jax
import jax.numpy as jnp
from jax import lax
from jax.experimental import pallas as pl
from jax.experimental.pallas import tpu as pltpu
from jax.experimental.pallas import tpu_sc as plsc

_N = 50000
_E = 800000
_H = 160
_R = 400

_NC = 2
_NS = 16
_NW = _NC * _NS
_B = 128
_G = 8
_NB0 = 232
_NB1 = 160
_TOTB = _NS * (_NB0 + _NB1)
_TOTB_PAD = 6432
_EPAD = _TOTB_PAD * _B
_C = 32
_NCHUNK = 5
_NPAD = 50176
_ZROWS = 3136


def _leaky(x):
    return jnp.where(x >= 0, x, 0.01 * x)


def _full(shape):
    return pl.BlockSpec(shape, lambda i: tuple(0 for _ in shape))


def _dense_body(np_ref, nc_ref, des_ref, tw_ref, px_ref,
                Wnp, bnp, Wnc, bnc, Wdes, bdes, Wtx, btx, Wtw, btw,
                Win, bin_, *out_refs):
    n = _leaky(np_ref[...] @ Wnp[...].T + bnp[...])
    c = _leaky(nc_ref[...] @ Wnc[...].T + bnc[...])
    d = _leaky(des_ref[...] @ Wdes[...].T + bdes[...])
    tw = _leaky(tw_ref[...] @ Wtx[...].T + btx[...])
    pt = _leaky(px_ref[...] @ Wtw[...].T + btw[...])
    h = jnp.concatenate([n, c, d, tw, pt], axis=1)
    h = _leaky(h @ Win[...].T + bin_[...])
    for k in range(_NCHUNK):
        out_refs[k][...] = h[:, k * _C:(k + 1) * _C]


def _dense_stage(num_prop, num_category, des_tensor, tweet_tensor, pre_x, p):
    grid = (_N // _R,)
    row = lambda shape: pl.BlockSpec((_R,) + shape[1:],
                                     lambda i: (i,) + tuple(0 for _ in shape[1:]))
    in_specs = [
        row((_N, 6)), row((_N, 11)), row((_N, 768)), row((_N, 768)), row((_N, 768)),
        _full((32, 6)), _full((1, 32)),
        _full((32, 11)), _full((1, 32)),
        _full((32, 768)), _full((1, 32)),
        _full((32, 768)), _full((1, 32)),
        _full((32, 768)), _full((1, 32)),
        _full((_H, _H)), _full((1, _H)),
    ]
    return pl.pallas_call(
        _dense_body,
        grid=grid,
        in_specs=in_specs,
        out_specs=[pl.BlockSpec((_R, _C), lambda i: (i, 0))] * _NCHUNK,
        out_shape=[jax.ShapeDtypeStruct((_N, _C), jnp.float32)] * _NCHUNK,
    )(num_prop, num_category, des_tensor, tweet_tensor, pre_x,
      p['W_np'], p['b_np'].reshape(1, 32),
      p['W_nc'], p['b_nc'].reshape(1, 32),
      p['W_des'], p['b_des'].reshape(1, 32),
      p['W_text'], p['b_text'].reshape(1, 32),
      p['W_tweet'], p['b_tweet'].reshape(1, 32),
      p['W_in'], p['b_in'].reshape(1, _H))


def _sc_mesh():
    return plsc.VectorSubcoreMesh(core_axis_name="c", subcore_axis_name="s",
                                  num_cores=_NC, num_subcores=_NS)
_SC_PARAMS = pltpu.CompilerParams(use_tc_tiling_on_sc=False)


def _sc_count(dst_r, ones_hbm, zeros_hbm):
    @functools.partial(
        pl.kernel,
        out_type=jax.ShapeDtypeStruct((_NC, _NPAD, 16), jnp.float32),
        mesh=_sc_mesh(),
        compiler_params=_SC_PARAMS,
        scratch_types=[
            pltpu.VMEM((_G, _B), jnp.int32),
            pltpu.VMEM((_B, 16), jnp.float32),
            pltpu.VMEM_SHARED((_NPAD, 16), jnp.float32),
        ],
    )
    def k(dst_hbm, ones_in, zeros_in, out_hbm, dst_g, ones, acc):
        cid = lax.axis_index("c")
        sid = lax.axis_index("s")
        ngr = jnp.where(cid == 0, _NB0 // _G, _NB1 // _G)
        start = jnp.where(cid == 0, sid * _NB0, _NS * _NB0 + sid * _NB1)
        pltpu.sync_copy(ones_in, ones)
        pltpu.sync_copy(zeros_in, acc.at[pl.ds(sid * _ZROWS, _ZROWS)])
        plsc.subcore_barrier()

        @pl.loop(0, ngr)
        def _(g):
            pltpu.sync_copy(dst_hbm.at[pl.ds(start + g * _G, _G)], dst_g)
            for j in range(_G):
                pltpu.sync_copy(ones, acc.at[dst_g.at[j]], add=True)

        plsc.subcore_barrier()
        pltpu.sync_copy(acc.at[pl.ds(sid * _ZROWS, _ZROWS)],
                        out_hbm.at[cid, pl.ds(sid * _ZROWS, _ZROWS)])

    return k(dst_r, ones_hbm, zeros_hbm)


def _sc_segsum(tables, src_r, dst_r, zeros_hbm):
    @functools.partial(
        pl.kernel,
        out_type=jax.ShapeDtypeStruct((_NC, _NCHUNK, _NPAD, _C), jnp.float32),
        mesh=_sc_mesh(),
        compiler_params=_SC_PARAMS,
        scratch_types=[
            pltpu.VMEM((_G, _B), jnp.int32),
            pltpu.VMEM((_G, _B), jnp.int32),
            pltpu.VMEM((_B, _C), jnp.float32),
            pltpu.VMEM((_B, _C), jnp.float32),
            pltpu.VMEM((_B, _C), jnp.float32),
            pltpu.VMEM((_B, _C), jnp.float32),
            pltpu.VMEM_SHARED((_NPAD, _C), jnp.float32),
            pltpu.SemaphoreType.DMA,
        ],
    )
    def k(t0, t1, t2, t3, t4, zeros_in, src_hbm, dst_hbm, out_hbm,
          src_g, dst_g, rb0, rb1, rb2, rb3, acc, sem):
        cid = lax.axis_index("c")
        sid = lax.axis_index("s")
        ngr = jnp.where(cid == 0, _NB0 // _G, _NB1 // _G)
        start = jnp.where(cid == 0, sid * _NB0, _NS * _NB0 + sid * _NB1)
        rbufs = (rb0, rb1, rb2, rb3)

        for c, tbl in enumerate((t0, t1, t2, t3, t4)):
            pltpu.sync_copy(zeros_in, acc.at[pl.ds(sid * _ZROWS, _ZROWS)])
            plsc.subcore_barrier()

            @pl.loop(0, ngr)
            def _(g):
                base = start + g * _G
                pltpu.sync_copy(src_hbm.at[pl.ds(base, _G)], src_g)
                pltpu.sync_copy(dst_hbm.at[pl.ds(base, _G)], dst_g)
                for j0 in range(0, _G, 4):
                    cps = [pltpu.async_copy(tbl.at[src_g.at[j0 + b]],
                                            rbufs[b], sem)
                           for b in range(4)]
                    for b in range(4):
                        cps[b].wait()
                        pltpu.sync_copy(rbufs[b], acc.at[dst_g.at[j0 + b]],
                                        add=True)

            plsc.subcore_barrier()
            pltpu.sync_copy(acc.at[pl.ds(sid * _ZROWS, _ZROWS)],
                            out_hbm.at[cid, c, pl.ds(sid * _ZROWS, _ZROWS)])
            plsc.subcore_barrier()

    return k(*tables, zeros_hbm, src_r, dst_r)


def _combine_body(parts_ref, cntp_ref, h0, h1, h2, h3, h4, Wl, bl, Wr,
                  *out_refs):
    p = parts_ref[...]
    agg = p[0] + p[1]
    agg = jnp.concatenate([agg[k] for k in range(_NCHUNK)], axis=1)
    cnt = cntp_ref[0, :, 0:1] + cntp_ref[1, :, 0:1]
    agg = agg / jnp.maximum(cnt, 1.0)
    h = jnp.concatenate([h0[...], h1[...], h2[...], h3[...], h4[...]], axis=1)
    out = agg @ Wl[...].T + bl[...] + h @ Wr[...].T
    for k in range(_NCHUNK):
        out_refs[k][...] = out[:, k * _C:(k + 1) * _C]


def _combine_stage(parts, cntp, hs, Wl, bl, Wr):
    grid = (_N // _R,)
    chunk = pl.BlockSpec((_R, _C), lambda i: (i, 0))
    return pl.pallas_call(
        _combine_body,
        grid=grid,
        in_specs=[
            pl.BlockSpec((_NC, _NCHUNK, _R, _C), lambda i: (0, 0, i, 0)),
            pl.BlockSpec((_NC, _R, 16), lambda i: (0, i, 0)),
            chunk, chunk, chunk, chunk, chunk,
            _full((_H, _H)), _full((1, _H)), _full((_H, _H)),
        ],
        out_specs=[chunk] * _NCHUNK,
        out_shape=[jax.ShapeDtypeStruct((_N, _C), jnp.float32)] * _NCHUNK,
    )(parts, cntp, *hs, Wl, bl.reshape(1, _H), Wr)


def _out_body(h0, h1, h2, h3, h4, Wo1, bo1, Wo2, bo2, out_ref, em_ref):
    h = jnp.concatenate([h0[...], h1[...], h2[...], h3[...], h4[...]], axis=1)
    em = _leaky(h @ Wo1[...].T + bo1[...])
    em_ref[...] = em
    out_ref[...] = em @ Wo2[...].T + bo2[...]


def _out_stage(hs, p):
    grid = (_N // _R,)
    chunk = pl.BlockSpec((_R, _C), lambda i: (i, 0))
    return pl.pallas_call(
        _out_body,
        grid=grid,
        in_specs=[
            chunk, chunk, chunk, chunk, chunk,
            _full((80, _H)), _full((1, 80)),
            _full((2, 80)), _full((1, 2)),
        ],
        out_specs=[
            pl.BlockSpec((_R, 2), lambda i: (i, 0)),
            pl.BlockSpec((_R, 80), lambda i: (i, 0)),
        ],
        out_shape=[
            jax.ShapeDtypeStruct((_N, 2), jnp.float32),
            jax.ShapeDtypeStruct((_N, 80), jnp.float32),
        ],
    )(*hs, p['W_o1'], p['b_o1'].reshape(1, 80), p['W_o2'], p['b_o2'].reshape(1, 2))


def kernel(pre_x, x, num_prop, num_category, des_tensor, tweet_tensor,
           edge_index, edge_type, params):
    p = params
    pad = _EPAD - _E
    src_r = jnp.concatenate([edge_index[0], jnp.zeros((pad,), jnp.int32)]
                            ).reshape(_TOTB_PAD, _B)
    dst_r = jnp.concatenate([edge_index[1], jnp.full((pad,), _N, jnp.int32)]
                            ).reshape(_TOTB_PAD, _B)
    ones16 = jnp.ones((_B, 16), jnp.float32)
    zeros16 = jnp.zeros((_ZROWS, 16), jnp.float32)
    zeros32 = jnp.zeros((_ZROWS, _C), jnp.float32)

    cntp = _sc_count(dst_r, ones16, zeros16)

    hs = _dense_stage(num_prop, num_category, des_tensor, tweet_tensor, pre_x, p)

    parts = _sc_segsum(hs, src_r, dst_r, zeros32)
    hs1 = _combine_stage(parts, cntp, hs, p['W1l'], p['b1l'], p['W1r'])

    parts = _sc_segsum(hs1, src_r, dst_r, zeros32)
    hs2 = _combine_stage(parts, cntp, hs1, p['W2l'], p['b2l'], p['W2r'])

    out, em = _out_stage(hs2, p)
    return (out, em)


# speedup vs baseline: 4.1960x; 1.0510x over previous
import functools

import jax
import jax.numpy as jnp
from jax import lax
from jax.experimental import pallas as pl
from jax.experimental.pallas import tpu as pltpu
from jax.experimental.pallas import tpu_sc as plsc

_N = 50000
_E = 800000
_H = 160
_R = 400

_NC = 2
_NS = 16
_NW = _NC * _NS
_B = 128
_G = 8
_NB0 = 232
_NB1 = 160
_TOTB = _NS * (_NB0 + _NB1)
_TOTB_PAD = 6432
_EPAD = _TOTB_PAD * _B
_C = 32
_NCHUNK = 5
_NPAD = 50176
_ZROWS = 3136


def _leaky(x):
    return jnp.where(x >= 0, x, 0.01 * x)


def _full(shape):
    return pl.BlockSpec(shape, lambda i: tuple(0 for _ in shape))


def _dense_body(np_ref, nc_ref, des_ref, tw_ref, px_ref,
                Wnp, bnp, Wnc, bnc, Wdes, bdes, Wtx, btx, Wtw, btw,
                Win, bin_, *out_refs):
    n = _leaky(np_ref[...] @ Wnp[...].T + bnp[...])
    c = _leaky(nc_ref[...] @ Wnc[...].T + bnc[...])
    d = _leaky(des_ref[...] @ Wdes[...].T + bdes[...])
    tw = _leaky(tw_ref[...] @ Wtx[...].T + btx[...])
    pt = _leaky(px_ref[...] @ Wtw[...].T + btw[...])
    h = jnp.concatenate([n, c, d, tw, pt], axis=1)
    h = _leaky(h @ Win[...].T + bin_[...])
    for k in range(_NCHUNK):
        out_refs[k][...] = h[:, k * _C:(k + 1) * _C]


def _dense_stage(num_prop, num_category, des_tensor, tweet_tensor, pre_x, p):
    grid = (_N // _R,)
    row = lambda shape: pl.BlockSpec((_R,) + shape[1:],
                                     lambda i: (i,) + tuple(0 for _ in shape[1:]))
    in_specs = [
        row((_N, 6)), row((_N, 11)), row((_N, 768)), row((_N, 768)), row((_N, 768)),
        _full((32, 6)), _full((1, 32)),
        _full((32, 11)), _full((1, 32)),
        _full((32, 768)), _full((1, 32)),
        _full((32, 768)), _full((1, 32)),
        _full((32, 768)), _full((1, 32)),
        _full((_H, _H)), _full((1, _H)),
    ]
    return pl.pallas_call(
        _dense_body,
        grid=grid,
        in_specs=in_specs,
        out_specs=[pl.BlockSpec((_R, _C), lambda i: (i, 0))] * _NCHUNK,
        out_shape=[jax.ShapeDtypeStruct((_N, _C), jnp.float32)] * _NCHUNK,
    )(num_prop, num_category, des_tensor, tweet_tensor, pre_x,
      p['W_np'], p['b_np'].reshape(1, 32),
      p['W_nc'], p['b_nc'].reshape(1, 32),
      p['W_des'], p['b_des'].reshape(1, 32),
      p['W_text'], p['b_text'].reshape(1, 32),
      p['W_tweet'], p['b_tweet'].reshape(1, 32),
      p['W_in'], p['b_in'].reshape(1, _H))


def _sc_mesh():
    return plsc.VectorSubcoreMesh(core_axis_name="c", subcore_axis_name="s",
                                  num_cores=_NC, num_subcores=_NS)
_SC_PARAMS = pltpu.CompilerParams(use_tc_tiling_on_sc=False)


def _sc_count(dst_r, ones_hbm, zeros_hbm):
    @functools.partial(
        pl.kernel,
        out_type=jax.ShapeDtypeStruct((_NC, _NPAD, 16), jnp.float32),
        mesh=_sc_mesh(),
        compiler_params=_SC_PARAMS,
        scratch_types=[
            pltpu.VMEM((_G, _B), jnp.int32),
            pltpu.VMEM((_B, 16), jnp.float32),
            pltpu.VMEM_SHARED((_NPAD, 16), jnp.float32),
        ],
    )
    def k(dst_hbm, ones_in, zeros_in, out_hbm, dst_g, ones, acc):
        cid = lax.axis_index("c")
        sid = lax.axis_index("s")
        ngr = jnp.where(cid == 0, _NB0 // _G, _NB1 // _G)
        start = jnp.where(cid == 0, sid * _NB0, _NS * _NB0 + sid * _NB1)
        pltpu.sync_copy(ones_in, ones)
        pltpu.sync_copy(zeros_in, acc.at[pl.ds(sid * _ZROWS, _ZROWS)])
        plsc.subcore_barrier()

        @pl.loop(0, ngr)
        def _(g):
            pltpu.sync_copy(dst_hbm.at[pl.ds(start + g * _G, _G)], dst_g)
            for j in range(_G):
                pltpu.sync_copy(ones, acc.at[dst_g.at[j]], add=True)

        plsc.subcore_barrier()
        pltpu.sync_copy(acc.at[pl.ds(sid * _ZROWS, _ZROWS)],
                        out_hbm.at[cid, pl.ds(sid * _ZROWS, _ZROWS)])

    return k(dst_r, ones_hbm, zeros_hbm)


def _sc_segsum(tables, src_r, dst_r, zeros_hbm):
    @functools.partial(
        pl.kernel,
        out_type=jax.ShapeDtypeStruct((_NC, _NCHUNK, _NPAD, _C), jnp.float32),
        mesh=_sc_mesh(),
        compiler_params=_SC_PARAMS,
        scratch_types=[
            pltpu.VMEM((_G, _B), jnp.int32),
            pltpu.VMEM((_G, _B), jnp.int32),
            pltpu.VMEM((_B, _C), jnp.float32),
            pltpu.VMEM((_B, _C), jnp.float32),
            pltpu.VMEM((_B, _C), jnp.float32),
            pltpu.VMEM((_B, _C), jnp.float32),
            pltpu.VMEM_SHARED((_NPAD, _C), jnp.float32),
            pltpu.SemaphoreType.DMA,
            pltpu.SemaphoreType.DMA,
        ],
    )
    def k(t0, t1, t2, t3, t4, zeros_in, src_hbm, dst_hbm, out_hbm,
          src_g, dst_g, rb0, rb1, rb2, rb3, acc, sem, sem2):
        cid = lax.axis_index("c")
        sid = lax.axis_index("s")
        ngr = jnp.where(cid == 0, _NB0 // _G, _NB1 // _G)
        start = jnp.where(cid == 0, sid * _NB0, _NS * _NB0 + sid * _NB1)
        rbufs = (rb0, rb1, rb2, rb3)

        for c, tbl in enumerate((t0, t1, t2, t3, t4)):
            pltpu.sync_copy(zeros_in, acc.at[pl.ds(sid * _ZROWS, _ZROWS)])
            plsc.subcore_barrier()

            @pl.loop(0, ngr)
            def _(g):
                base = start + g * _G
                pltpu.sync_copy(src_hbm.at[pl.ds(base, _G)], src_g)
                pltpu.sync_copy(dst_hbm.at[pl.ds(base, _G)], dst_g)
                cps = [pltpu.async_copy(tbl.at[src_g.at[b]], rbufs[b], sem)
                       for b in range(4)]
                scs = []
                for b in range(4):
                    cps[b].wait()
                    scs.append(pltpu.async_copy(rbufs[b],
                                                acc.at[dst_g.at[b]],
                                                sem2, add=True))
                cps2 = []
                for b in range(4):
                    scs[b].wait()
                    cps2.append(pltpu.async_copy(tbl.at[src_g.at[4 + b]],
                                                 rbufs[b], sem))
                scs2 = []
                for b in range(4):
                    cps2[b].wait()
                    scs2.append(pltpu.async_copy(rbufs[b],
                                                 acc.at[dst_g.at[4 + b]],
                                                 sem2, add=True))
                for s in scs2:
                    s.wait()

            plsc.subcore_barrier()
            pltpu.sync_copy(acc.at[pl.ds(sid * _ZROWS, _ZROWS)],
                            out_hbm.at[cid, c, pl.ds(sid * _ZROWS, _ZROWS)])
            plsc.subcore_barrier()

    return k(*tables, zeros_hbm, src_r, dst_r)


def _combine_body(parts_ref, cntp_ref, h0, h1, h2, h3, h4, Wl, bl, Wr,
                  *out_refs):
    p = parts_ref[...]
    agg = p[0] + p[1]
    agg = jnp.concatenate([agg[k] for k in range(_NCHUNK)], axis=1)
    cnt = cntp_ref[0, :, 0:1] + cntp_ref[1, :, 0:1]
    agg = agg / jnp.maximum(cnt, 1.0)
    h = jnp.concatenate([h0[...], h1[...], h2[...], h3[...], h4[...]], axis=1)
    out = agg @ Wl[...].T + bl[...] + h @ Wr[...].T
    for k in range(_NCHUNK):
        out_refs[k][...] = out[:, k * _C:(k + 1) * _C]


def _combine_stage(parts, cntp, hs, Wl, bl, Wr):
    grid = (_N // _R,)
    chunk = pl.BlockSpec((_R, _C), lambda i: (i, 0))
    return pl.pallas_call(
        _combine_body,
        grid=grid,
        in_specs=[
            pl.BlockSpec((_NC, _NCHUNK, _R, _C), lambda i: (0, 0, i, 0)),
            pl.BlockSpec((_NC, _R, 16), lambda i: (0, i, 0)),
            chunk, chunk, chunk, chunk, chunk,
            _full((_H, _H)), _full((1, _H)), _full((_H, _H)),
        ],
        out_specs=[chunk] * _NCHUNK,
        out_shape=[jax.ShapeDtypeStruct((_N, _C), jnp.float32)] * _NCHUNK,
    )(parts, cntp, *hs, Wl, bl.reshape(1, _H), Wr)


def _out_body(h0, h1, h2, h3, h4, Wo1, bo1, Wo2, bo2, out_ref, em_ref):
    h = jnp.concatenate([h0[...], h1[...], h2[...], h3[...], h4[...]], axis=1)
    em = _leaky(h @ Wo1[...].T + bo1[...])
    em_ref[...] = em
    out_ref[...] = em @ Wo2[...].T + bo2[...]


def _out_stage(hs, p):
    grid = (_N // _R,)
    chunk = pl.BlockSpec((_R, _C), lambda i: (i, 0))
    return pl.pallas_call(
        _out_body,
        grid=grid,
        in_specs=[
            chunk, chunk, chunk, chunk, chunk,
            _full((80, _H)), _full((1, 80)),
            _full((2, 80)), _full((1, 2)),
        ],
        out_specs=[
            pl.BlockSpec((_R, 2), lambda i: (i, 0)),
            pl.BlockSpec((_R, 80), lambda i: (i, 0)),
        ],
        out_shape=[
            jax.ShapeDtypeStruct((_N, 2), jnp.float32),
            jax.ShapeDtypeStruct((_N, 80), jnp.float32),
        ],
    )(*hs, p['W_o1'], p['b_o1'].reshape(1, 80), p['W_o2'], p['b_o2'].reshape(1, 2))


def kernel(pre_x, x, num_prop, num_category, des_tensor, tweet_tensor,
           edge_index, edge_type, params):
    p = params
    pad = _EPAD - _E
    src_r = jnp.concatenate([edge_index[0], jnp.zeros((pad,), jnp.int32)]
                            ).reshape(_TOTB_PAD, _B)
    dst_r = jnp.concatenate([edge_index[1], jnp.full((pad,), _N, jnp.int32)]
                            ).reshape(_TOTB_PAD, _B)
    ones16 = jnp.ones((_B, 16), jnp.float32)
    zeros16 = jnp.zeros((_ZROWS, 16), jnp.float32)
    zeros32 = jnp.zeros((_ZROWS, _C), jnp.float32)

    cntp = _sc_count(dst_r, ones16, zeros16)

    hs = _dense_stage(num_prop, num_category, des_tensor, tweet_tensor, pre_x, p)

    parts = _sc_segsum(hs, src_r, dst_r, zeros32)
    hs1 = _combine_stage(parts, cntp, hs, p['W1l'], p['b1l'], p['W1r'])

    parts = _sc_segsum(hs1, src_r, dst_r, zeros32)
    hs2 = _combine_stage(parts, cntp, hs1, p['W2l'], p['b2l'], p['W2r'])

    out, em = _out_stage(hs2, p)
    return (out, em)
